# baseline (speedup 1.0000x reference)
import functools

import jax
import jax.numpy as jnp
from jax import lax
from jax.experimental import pallas as pl
from jax.experimental.pallas import tpu as pltpu
from jax.experimental.pallas import tpu_sc as plsc

_NC = 2
_NS = 16
_NW = _NC * _NS
_LANES = 16

_TOTAL = 16384 * 2048
_SPAN = _TOTAL // _NW
_CHUNK = 16384
_NCHUNK = _SPAN // _CHUNK


def _relu_tile(x_hbm, o_hbm, buf, in_sem, out_sem):
    wid = lax.axis_index("s") * _NC + lax.axis_index("c")
    base = wid * _SPAN

    def chunk_body(i, carry):
        off = base + i * _CHUNK
        pltpu.async_copy(x_hbm.at[pl.ds(off, _CHUNK)], buf, in_sem).wait()

        def vec_body(j, c):
            sl = pl.ds(j * _LANES, _LANES)
            buf[sl] = jnp.maximum(buf[sl], 0.0)
            return c

        lax.fori_loop(0, _CHUNK // _LANES, vec_body, 0, unroll=8)
        pltpu.async_copy(buf, o_hbm.at[pl.ds(off, _CHUNK)], out_sem).wait()
        return carry

    lax.fori_loop(0, _NCHUNK, chunk_body, 0)


@functools.partial(jax.jit, static_argnames=())
def _sc_relu(xf):
    mesh = plsc.VectorSubcoreMesh(core_axis_name="c", subcore_axis_name="s")
    return pl.kernel(
        _relu_tile,
        out_type=jax.ShapeDtypeStruct((_TOTAL,), jnp.float32),
        mesh=mesh,
        scratch_types=[
            pltpu.VMEM((_CHUNK,), jnp.float32),
            pltpu.SemaphoreType.DMA,
            pltpu.SemaphoreType.DMA,
        ],
    )(xf)


def kernel(input):
    m, n = input.shape
    return _sc_relu(input.reshape(-1)).reshape(m, n)


# speedup vs baseline: 1.2221x; 1.2221x over previous
import functools

import jax
import jax.numpy as jnp
from jax import lax
from jax.experimental import pallas as pl
from jax.experimental.pallas import tpu as pltpu
from jax.experimental.pallas import tpu_sc as plsc

_NC = 2
_NS = 16
_NW = _NC * _NS
_LANES = 16

_TOTAL = 16384 * 2048
_SPAN = _TOTAL // _NW
_CHUNK = 16384
_NCHUNK = _SPAN // _CHUNK
_NBUF = 3


def _relu_tile(x_hbm, o_hbm, *scratch):
    in_bufs = scratch[0:_NBUF]
    out_bufs = scratch[_NBUF:2 * _NBUF]
    in_sems = scratch[2 * _NBUF:3 * _NBUF]
    out_sems = scratch[3 * _NBUF:4 * _NBUF]

    wid = lax.axis_index("s") * _NC + lax.axis_index("c")
    base = wid * _SPAN

    for b in range(_NBUF):
        pltpu.async_copy(
            x_hbm.at[pl.ds(base + b * _CHUNK, _CHUNK)], in_bufs[b],
            in_sems[b])

    for c in range(_NCHUNK):
        b = c % _NBUF
        src = x_hbm.at[pl.ds(base + c * _CHUNK, _CHUNK)]
        pltpu.make_async_copy(src, in_bufs[b], in_sems[b]).wait()
        if c >= _NBUF:
            prev = o_hbm.at[pl.ds(base + (c - _NBUF) * _CHUNK, _CHUNK)]
            pltpu.make_async_copy(out_bufs[b], prev, out_sems[b]).wait()

        ib = in_bufs[b]
        ob = out_bufs[b]

        @plsc.parallel_loop(0, _CHUNK // _LANES, unroll=8)
        def _(j):
            sl = pl.ds(j * _LANES, _LANES)
            ob[sl] = jnp.maximum(ib[sl], 0.0)

        pltpu.async_copy(
            out_bufs[b], o_hbm.at[pl.ds(base + c * _CHUNK, _CHUNK)],
            out_sems[b])
        if c + _NBUF < _NCHUNK:
            nxt = x_hbm.at[pl.ds(base + (c + _NBUF) * _CHUNK, _CHUNK)]
            pltpu.async_copy(nxt, in_bufs[b], in_sems[b])

    for c in range(_NCHUNK - _NBUF, _NCHUNK):
        b = c % _NBUF
        dst = o_hbm.at[pl.ds(base + c * _CHUNK, _CHUNK)]
        pltpu.make_async_copy(out_bufs[b], dst, out_sems[b]).wait()


@jax.jit
def _sc_relu(xf):
    mesh = plsc.VectorSubcoreMesh(core_axis_name="c", subcore_axis_name="s")
    return pl.kernel(
        _relu_tile,
        out_type=jax.ShapeDtypeStruct((_TOTAL,), jnp.float32),
        mesh=mesh,
        scratch_types=(
            [pltpu.VMEM((_CHUNK,), jnp.float32) for _ in range(2 * _NBUF)]
            + [pltpu.SemaphoreType.DMA for _ in range(2 * _NBUF)]
        ),
    )(xf)


def kernel(input):
    m, n = input.shape
    return _sc_relu(input.reshape(-1)).reshape(m, n)


# speedup vs baseline: 3.7301x; 3.0523x over previous
import jax
import jax.numpy as jnp
from jax import lax
from jax.experimental import pallas as pl
from jax.experimental.pallas import tpu as pltpu
from jax.experimental.pallas import tpu_sc as plsc

_NC = 2
_NS = 16
_NW = _NC * _NS
_LANES = 16

_M, _N = 16384, 2048
_ROWS_PER_TILE = _M // _NW
_CROWS = 8
_NCHUNK = _ROWS_PER_TILE // _CROWS
_NBUF = 4
_NGRP = _NCHUNK // _NBUF


def _relu_tile(x_hbm, o_hbm, *scratch):
    bufs = scratch[0:_NBUF]
    in_sems = scratch[_NBUF:2 * _NBUF]
    out_sems = scratch[2 * _NBUF:3 * _NBUF]

    wid = lax.axis_index("s") * _NC + lax.axis_index("c")
    base = wid * _ROWS_PER_TILE

    def rows(c):
        return pl.ds(base + c * _CROWS, _CROWS)

    for c0 in range(2):
        pltpu.async_copy(x_hbm.at[rows(c0)], bufs[c0], in_sems[c0])

    def body(g, carry):
        for b in range(_NBUF):
            c = g * _NBUF + b
            bt = (b + 2) % _NBUF

            @pl.when(c + 2 < _NCHUNK)
            def _prefetch():
                @pl.when(c >= 2)
                def _drain():
                    pltpu.make_async_copy(
                        bufs[bt], o_hbm.at[rows(c - 2)], out_sems[bt]).wait()
                pltpu.async_copy(x_hbm.at[rows(c + 2)], bufs[bt], in_sems[bt])

            pltpu.make_async_copy(x_hbm.at[rows(c)], bufs[b],
                                  in_sems[b]).wait()

            buf = bufs[b]
            for r in range(_CROWS):
                @plsc.parallel_loop(0, _N // _LANES, unroll=8)
                def _(j):
                    sl = pl.ds(j * _LANES, _LANES)
                    buf[r, sl] = jnp.maximum(buf[r, sl], 0.0)

            pltpu.async_copy(bufs[b], o_hbm.at[rows(c)], out_sems[b])
        return carry

    lax.fori_loop(0, _NGRP, body, 0)

    for c in range(_NCHUNK - _NBUF, _NCHUNK):
        b = c % _NBUF
        pltpu.make_async_copy(bufs[b], o_hbm.at[rows(c)], out_sems[b]).wait()


@jax.jit
def _sc_relu(x):
    mesh = plsc.VectorSubcoreMesh(core_axis_name="c", subcore_axis_name="s")
    return pl.kernel(
        _relu_tile,
        out_type=jax.ShapeDtypeStruct((_M, _N), jnp.float32),
        mesh=mesh,
        scratch_types=(
            [pltpu.VMEM((_CROWS, _N), jnp.float32) for _ in range(_NBUF)]
            + [pltpu.SemaphoreType.DMA for _ in range(2 * _NBUF)]
        ),
    )(x)


def kernel(input):
    return _sc_relu(input)
